# baseline (speedup 1.0000x reference)
import functools

import jax
import jax.numpy as jnp
from jax import lax
from jax.experimental import pallas as pl
from jax.experimental.pallas import tpu as pltpu
from jax.experimental.pallas import tpu_sc as plsc

B, N, C, OUT = 4, 4096, 64, 64
K = 21
KP = 24
TR = 256
TN = 512
NW = 32
PW = (B * N) // NW
GP = 4
GI = GP * KP
NG = PW // GP
NCH = 2 * OUT
CNT = B * N * K
EPS = 1e-5


def _knn_proj_body(xyz_ref, xyzt_ref, f_ref, wg_ref, wf_ref,
                   idx_ref, r_ref, c_ref, d2_ref):
    b = pl.program_id(0)
    t = pl.program_id(1)

    xt = xyz_ref[0]
    xft = xyzt_ref[0]

    sqt = jnp.sum(xt * xt, axis=1, keepdims=True)
    sqf = jnp.sum(xft * xft, axis=0)[None, :]
    cross = lax.dot_general(xt, xft, (((1,), (0,)), ((), ())),
                            preferred_element_type=jnp.float32)
    d2_ref[...] = sqt + sqf - 2.0 * cross

    rows = lax.broadcasted_iota(jnp.int32, (TR, KP), 0) + t * TR

    iota = lax.broadcasted_iota(jnp.int32, (TR, N), 1)
    kcol = lax.broadcasted_iota(jnp.int32, (TR, KP), 1)
    inf = jnp.float32(jnp.inf)

    def step(k, acc):
        v = d2_ref[...]
        m = jnp.min(v, axis=1, keepdims=True)
        sel = jnp.where(v == m, iota, jnp.int32(N))
        i = jnp.min(sel, axis=1, keepdims=True)
        acc = jnp.where(kcol == k, i, acc)
        d2_ref[...] = jnp.where(sel == i, inf, v)
        return acc

    idx_final = lax.fori_loop(0, K, step, rows)
    idx_ref[0] = idx_final + b * N

    wg = wg_ref[...]
    a = wg[:, 0:3]
    d = wg[:, 3:6]
    rgeo = lax.dot_general(xt, d, (((1,), (1,)), ((), ())),
                           preferred_element_type=jnp.float32)
    cgeo = lax.dot_general(xt, a - d, (((1,), (1,)), ((), ())),
                           preferred_element_type=jnp.float32)
    fb = f_ref[0]
    wf = wf_ref[...]
    w1 = wf[:, 0:C]
    w2 = wf[:, C:2 * C]
    rfeat = lax.dot_general(fb, w1, (((0,), (1,)), ((), ())),
                            preferred_element_type=jnp.float32)
    cfeat = lax.dot_general(fb, w2, (((0,), (1,)), ((), ())),
                            preferred_element_type=jnp.float32)
    r_ref[...] = jnp.concatenate([rgeo, rfeat], axis=1)
    c_ref[...] = jnp.concatenate([cgeo, cfeat], axis=1)


def _knn_proj(xyz, xyzt, f, w_geo, w_feat):
    nt = N // TR
    return pl.pallas_call(
        _knn_proj_body,
        grid=(B, nt),
        in_specs=[
            pl.BlockSpec((1, TR, 3), lambda b, t: (b, t, 0)),
            pl.BlockSpec((1, 3, N), lambda b, t: (b, 0, 0)),
            pl.BlockSpec((1, C, TR), lambda b, t: (b, 0, t)),
            pl.BlockSpec((OUT, 6), lambda b, t: (0, 0)),
            pl.BlockSpec((OUT, 2 * C), lambda b, t: (0, 0)),
        ],
        out_specs=[
            pl.BlockSpec((1, TR, KP), lambda b, t: (b, t, 0)),
            pl.BlockSpec((TR, NCH), lambda b, t: (b * nt + t, 0)),
            pl.BlockSpec((TR, NCH), lambda b, t: (b * nt + t, 0)),
        ],
        out_shape=[
            jax.ShapeDtypeStruct((B, N, KP), jnp.int32),
            jax.ShapeDtypeStruct((B * N, NCH), jnp.float32),
            jax.ShapeDtypeStruct((B * N, NCH), jnp.float32),
        ],
        scratch_shapes=[
            pltpu.VMEM((TR, N), jnp.float32),
        ],
    )(xyz, xyzt, f, w_geo, w_feat)


def _gather_reduce_body(table_ref, idxg_ref, ctab_ref, m_ref, stats_ref,
                        idx_v, rows_v, cbuf, mbuf, acc, sems):
    cid = lax.axis_index("c")
    sid = lax.axis_index("s")
    wid = sid * 2 + cid
    gbase = wid * NG
    pbase = wid * PW

    pltpu.sync_copy(idxg_ref.at[pl.ds(gbase, NG)], idx_v)

    zeros = jnp.zeros((16,), jnp.float32)
    for i in range(5):
        for cch in range(NCH // 16):
            acc[i, pl.ds(cch * 16, 16)] = zeros

    def process(g, buf):
        pltpu.sync_copy(ctab_ref.at[pl.ds(pbase + g * GP, GP)], cbuf)
        pltpu.make_async_copy(
            table_ref.at[idx_v.at[g]], rows_v.at[buf], sems.at[buf]).wait()

        for p in range(GP):
            base = p * KP
            m0 = [rows_v[buf, base, pl.ds(cch * 16, 16)]
                  for cch in range(NCH // 16)]
            s0 = list(m0)
            q0 = [v * v for v in m0]

            def kstep(k, carry):
                ms, ss, qs = carry
                nm, ns, nq = [], [], []
                for cch in range(NCH // 16):
                    v = rows_v[buf, base + k, pl.ds(cch * 16, 16)]
                    nm.append(jnp.maximum(ms[cch], v))
                    ns.append(ss[cch] + v)
                    nq.append(qs[cch] + v * v)
                return tuple(nm), tuple(ns), tuple(nq)

            ms, ss, qs = lax.fori_loop(
                1, K, kstep, (tuple(m0), tuple(s0), tuple(q0)))

            for cch in range(NCH // 16):
                sl = pl.ds(cch * 16, 16)
                cv = cbuf[p, sl]
                mbuf[p, sl] = cv + ms[cch]
                acc[0, sl] = acc[0, sl] + ss[cch]
                acc[1, sl] = acc[1, sl] + qs[cch]
                acc[2, sl] = acc[2, sl] + cv * ss[cch]
                acc[3, sl] = acc[3, sl] + cv
                acc[4, sl] = acc[4, sl] + cv * cv

        pltpu.sync_copy(mbuf, m_ref.at[pl.ds(pbase + g * GP, GP)])

    def fire(g, buf):
        pltpu.async_copy(table_ref.at[idx_v.at[g]], rows_v.at[buf], sems.at[buf])

    fire(0, 0)

    def pair(h, _):
        g0 = 2 * h
        fire(g0 + 1, 1)
        process(g0, 0)

        @pl.when(g0 + 2 < NG)
        def _():
            fire(g0 + 2, 0)

        process(g0 + 1, 1)
        return 0

    lax.fori_loop(0, NG // 2, pair, 0)
    pltpu.sync_copy(acc, stats_ref.at[wid])


def _gather_reduce(table, idxg, ctab):
    mesh = plsc.VectorSubcoreMesh(
        core_axis_name="c", subcore_axis_name="s",
        num_cores=2, num_subcores=16)
    kern = pl.kernel(
        _gather_reduce_body,
        out_type=[
            jax.ShapeDtypeStruct((B * N, NCH), jnp.float32),
            jax.ShapeDtypeStruct((NW, 5, NCH), jnp.float32),
        ],
        mesh=mesh,
        scratch_types=[
            pltpu.VMEM((NG, GI), jnp.int32),
            pltpu.VMEM((2, GI, NCH), jnp.float32),
            pltpu.VMEM((GP, NCH), jnp.float32),
            pltpu.VMEM((GP, NCH), jnp.float32),
            pltpu.VMEM((5, NCH), jnp.float32),
            pltpu.SemaphoreType.DMA((2,)),
        ],
    )
    return kern(table, idxg, ctab)


def _finalize_body(m_ref, stats_ref, g_ref, bt_ref, out_ref):
    st = stats_ref[...]
    tot = jnp.sum(st, axis=0)
    a1, a2, a3, a4, a5 = tot[0], tot[1], tot[2], tot[3], tot[4]
    kf = jnp.float32(K)
    inv = jnp.float32(1.0 / CNT)
    mean = (kf * a4 + a1) * inv
    e2 = (kf * a5 + 2.0 * a3 + a2) * inv
    var = e2 - mean * mean
    scale = g_ref[0] * lax.rsqrt(var + jnp.float32(EPS))
    shift = bt_ref[0] - mean * scale
    y = jnp.maximum(m_ref[...] * scale[None, :] + shift[None, :], 0.0)
    out_ref[0] = y.T


def _finalize(m, stats, gamma, beta):
    nt = N // TN
    return pl.pallas_call(
        _finalize_body,
        grid=(B, nt),
        in_specs=[
            pl.BlockSpec((TN, NCH), lambda b, t: (b * nt + t, 0)),
            pl.BlockSpec((NW, 5, NCH), lambda b, t: (0, 0, 0)),
            pl.BlockSpec((1, NCH), lambda b, t: (0, 0)),
            pl.BlockSpec((1, NCH), lambda b, t: (0, 0)),
        ],
        out_specs=pl.BlockSpec((1, NCH, TN), lambda b, t: (b, 0, t)),
        out_shape=jax.ShapeDtypeStruct((B, NCH, N), jnp.float32),
    )(m, stats, gamma, beta)


@jax.jit
def kernel(xyz, f, W_geo, gamma_geo, beta_geo, W_feat, gamma_feat, beta_feat):
    xyzt = jnp.transpose(xyz, (0, 2, 1))
    idx, rtab, ctab = _knn_proj(xyz, xyzt, f, W_geo, W_feat)
    idxg = idx.reshape((B * N) // GP, GI)
    m, stats = _gather_reduce(rtab, idxg, ctab)
    gamma = jnp.concatenate([gamma_geo, gamma_feat]).reshape(1, NCH)
    beta = jnp.concatenate([beta_geo, beta_feat]).reshape(1, NCH)
    return _finalize(m, stats, gamma, beta)


# speedup vs baseline: 1.0942x; 1.0942x over previous
import functools

import jax
import jax.numpy as jnp
from jax import lax
from jax.experimental import pallas as pl
from jax.experimental.pallas import tpu as pltpu
from jax.experimental.pallas import tpu_sc as plsc

B, N, C, OUT = 4, 4096, 64, 64
K = 21
KP = 24
TR = 256
TN = 512
NW = 32
PW = (B * N) // NW
GP = 4
GI = GP * KP
NG = PW // GP
NCH = 2 * OUT
CNT = B * N * K
EPS = 1e-5


def _knn_proj_body(xyz_ref, xyzt_ref, f_ref, wg_ref, wf_ref,
                   idx_ref, r_ref, c_ref, d2_ref):
    b = pl.program_id(0)
    t = pl.program_id(1)

    xt = xyz_ref[0]
    xft = xyzt_ref[0]

    sqt = jnp.sum(xt * xt, axis=1, keepdims=True)
    sqf = jnp.sum(xft * xft, axis=0)[None, :]
    cross = lax.dot_general(xt, xft, (((1,), (0,)), ((), ())),
                            preferred_element_type=jnp.float32)
    d2_ref[...] = sqt + sqf - 2.0 * cross

    rows = lax.broadcasted_iota(jnp.int32, (TR, KP), 0) + t * TR

    iota = lax.broadcasted_iota(jnp.int32, (TR, N), 1)
    kcol = lax.broadcasted_iota(jnp.int32, (TR, KP), 1)
    inf = jnp.float32(jnp.inf)

    def step(k, acc):
        v = d2_ref[...]
        m = jnp.min(v, axis=1, keepdims=True)
        sel = jnp.where(v == m, iota, jnp.int32(N))
        i = jnp.min(sel, axis=1, keepdims=True)
        acc = jnp.where(kcol == k, i, acc)
        d2_ref[...] = jnp.where(iota == i, inf, v)
        return acc

    idx_final = lax.fori_loop(0, K, step, rows)
    idx_ref[0] = idx_final + b * N

    wg = wg_ref[...]
    a = wg[:, 0:3]
    d = wg[:, 3:6]
    rgeo = lax.dot_general(xt, d, (((1,), (1,)), ((), ())),
                           preferred_element_type=jnp.float32)
    cgeo = lax.dot_general(xt, a - d, (((1,), (1,)), ((), ())),
                           preferred_element_type=jnp.float32)
    fb = f_ref[0]
    wf = wf_ref[...]
    w1 = wf[:, 0:C]
    w2 = wf[:, C:2 * C]
    rfeat = lax.dot_general(fb, w1, (((0,), (1,)), ((), ())),
                            preferred_element_type=jnp.float32)
    cfeat = lax.dot_general(fb, w2, (((0,), (1,)), ((), ())),
                            preferred_element_type=jnp.float32)
    r_ref[...] = jnp.concatenate([rgeo, rfeat], axis=1)
    c_ref[...] = jnp.concatenate([cgeo, cfeat], axis=1)


def _knn_proj(xyz, xyzt, f, w_geo, w_feat):
    nt = N // TR
    return pl.pallas_call(
        _knn_proj_body,
        grid=(B, nt),
        in_specs=[
            pl.BlockSpec((1, TR, 3), lambda b, t: (b, t, 0)),
            pl.BlockSpec((1, 3, N), lambda b, t: (b, 0, 0)),
            pl.BlockSpec((1, C, TR), lambda b, t: (b, 0, t)),
            pl.BlockSpec((OUT, 6), lambda b, t: (0, 0)),
            pl.BlockSpec((OUT, 2 * C), lambda b, t: (0, 0)),
        ],
        out_specs=[
            pl.BlockSpec((1, TR, KP), lambda b, t: (b, t, 0)),
            pl.BlockSpec((TR, NCH), lambda b, t: (b * nt + t, 0)),
            pl.BlockSpec((TR, NCH), lambda b, t: (b * nt + t, 0)),
        ],
        out_shape=[
            jax.ShapeDtypeStruct((B, N, KP), jnp.int32),
            jax.ShapeDtypeStruct((B * N, NCH), jnp.float32),
            jax.ShapeDtypeStruct((B * N, NCH), jnp.float32),
        ],
        scratch_shapes=[
            pltpu.VMEM((TR, N), jnp.float32),
        ],
    )(xyz, xyzt, f, w_geo, w_feat)


def _gather_reduce_body(table_ref, idxg_ref, ctab_ref, m_ref, stats_ref,
                        idx_v, rows_v, cbuf, mbuf, acc, sems):
    cid = lax.axis_index("c")
    sid = lax.axis_index("s")
    wid = sid * 2 + cid
    gbase = wid * NG
    pbase = wid * PW

    pltpu.sync_copy(idxg_ref.at[pl.ds(gbase, NG)], idx_v)

    zeros = jnp.zeros((16,), jnp.float32)
    for i in range(5):
        for cch in range(NCH // 16):
            acc[i, pl.ds(cch * 16, 16)] = zeros

    def process(g, buf):
        pltpu.sync_copy(ctab_ref.at[pl.ds(pbase + g * GP, GP)], cbuf)
        pltpu.make_async_copy(
            table_ref.at[idx_v.at[g]], rows_v.at[buf], sems.at[buf]).wait()

        for p in range(GP):
            base = p * KP
            m0 = [rows_v[buf, base, pl.ds(cch * 16, 16)]
                  for cch in range(NCH // 16)]
            s0 = list(m0)
            q0 = [v * v for v in m0]

            def kstep(k, carry):
                ms, ss, qs = carry
                nm, ns, nq = [], [], []
                for cch in range(NCH // 16):
                    v = rows_v[buf, base + k, pl.ds(cch * 16, 16)]
                    nm.append(jnp.maximum(ms[cch], v))
                    ns.append(ss[cch] + v)
                    nq.append(qs[cch] + v * v)
                return tuple(nm), tuple(ns), tuple(nq)

            ms, ss, qs = lax.fori_loop(
                1, K, kstep, (tuple(m0), tuple(s0), tuple(q0)))

            for cch in range(NCH // 16):
                sl = pl.ds(cch * 16, 16)
                cv = cbuf[p, sl]
                mbuf[p, sl] = cv + ms[cch]
                acc[0, sl] = acc[0, sl] + ss[cch]
                acc[1, sl] = acc[1, sl] + qs[cch]
                acc[2, sl] = acc[2, sl] + cv * ss[cch]
                acc[3, sl] = acc[3, sl] + cv
                acc[4, sl] = acc[4, sl] + cv * cv

        pltpu.sync_copy(mbuf, m_ref.at[pl.ds(pbase + g * GP, GP)])

    def fire(g, buf):
        pltpu.async_copy(table_ref.at[idx_v.at[g]], rows_v.at[buf], sems.at[buf])

    fire(0, 0)

    def pair(h, _):
        g0 = 2 * h
        fire(g0 + 1, 1)
        process(g0, 0)

        @pl.when(g0 + 2 < NG)
        def _():
            fire(g0 + 2, 0)

        process(g0 + 1, 1)
        return 0

    lax.fori_loop(0, NG // 2, pair, 0)
    pltpu.sync_copy(acc, stats_ref.at[wid])


def _gather_reduce(table, idxg, ctab):
    mesh = plsc.VectorSubcoreMesh(
        core_axis_name="c", subcore_axis_name="s",
        num_cores=2, num_subcores=16)
    kern = pl.kernel(
        _gather_reduce_body,
        out_type=[
            jax.ShapeDtypeStruct((B * N, NCH), jnp.float32),
            jax.ShapeDtypeStruct((NW, 5, NCH), jnp.float32),
        ],
        mesh=mesh,
        scratch_types=[
            pltpu.VMEM((NG, GI), jnp.int32),
            pltpu.VMEM((2, GI, NCH), jnp.float32),
            pltpu.VMEM((GP, NCH), jnp.float32),
            pltpu.VMEM((GP, NCH), jnp.float32),
            pltpu.VMEM((5, NCH), jnp.float32),
            pltpu.SemaphoreType.DMA((2,)),
        ],
    )
    return kern(table, idxg, ctab)


def _finalize_body(m_ref, stats_ref, g_ref, bt_ref, out_ref):
    st = stats_ref[...]
    tot = jnp.sum(st, axis=0)
    a1, a2, a3, a4, a5 = tot[0], tot[1], tot[2], tot[3], tot[4]
    kf = jnp.float32(K)
    inv = jnp.float32(1.0 / CNT)
    mean = (kf * a4 + a1) * inv
    e2 = (kf * a5 + 2.0 * a3 + a2) * inv
    var = e2 - mean * mean
    scale = g_ref[0] * lax.rsqrt(var + jnp.float32(EPS))
    shift = bt_ref[0] - mean * scale
    y = jnp.maximum(m_ref[...] * scale[None, :] + shift[None, :], 0.0)
    out_ref[0] = y.T


def _finalize(m, stats, gamma, beta):
    nt = N // TN
    return pl.pallas_call(
        _finalize_body,
        grid=(B, nt),
        in_specs=[
            pl.BlockSpec((TN, NCH), lambda b, t: (b * nt + t, 0)),
            pl.BlockSpec((NW, 5, NCH), lambda b, t: (0, 0, 0)),
            pl.BlockSpec((1, NCH), lambda b, t: (0, 0)),
            pl.BlockSpec((1, NCH), lambda b, t: (0, 0)),
        ],
        out_specs=pl.BlockSpec((1, NCH, TN), lambda b, t: (b, 0, t)),
        out_shape=jax.ShapeDtypeStruct((B, NCH, N), jnp.float32),
    )(m, stats, gamma, beta)


@jax.jit
def kernel(xyz, f, W_geo, gamma_geo, beta_geo, W_feat, gamma_feat, beta_feat):
    xyzt = jnp.transpose(xyz, (0, 2, 1))
    idx, rtab, ctab = _knn_proj(xyz, xyzt, f, W_geo, W_feat)
    idxg = idx.reshape((B * N) // GP, GI)
    m, stats = _gather_reduce(rtab, idxg, ctab)
    gamma = jnp.concatenate([gamma_geo, gamma_feat]).reshape(1, NCH)
    beta = jnp.concatenate([beta_geo, beta_feat]).reshape(1, NCH)
    return _finalize(m, stats, gamma, beta)


# speedup vs baseline: 1.2519x; 1.1441x over previous
import functools

import jax
import jax.numpy as jnp
from jax import lax
from jax.experimental import pallas as pl
from jax.experimental.pallas import tpu as pltpu
from jax.experimental.pallas import tpu_sc as plsc

B, N, C, OUT = 4, 4096, 64, 64
K = 21
KP = 24
TR = 256
TN = 512
NW = 32
PW = (B * N) // NW
GP = 4
GI = GP * KP
NG = PW // GP
NCH = 2 * OUT
CNT = B * N * K
EPS = 1e-5


def _knn_proj_body(xyz_ref, xyzt_ref, f_ref, wg_ref, wf_ref,
                   idx_ref, r_ref, c_ref, d2_ref):
    b = pl.program_id(0)
    t = pl.program_id(1)

    xt = xyz_ref[0]
    xft = xyzt_ref[0]

    sqt = jnp.sum(xt * xt, axis=1, keepdims=True)
    sqf = jnp.sum(xft * xft, axis=0)[None, :]
    cross = lax.dot_general(xt, xft, (((1,), (0,)), ((), ())),
                            preferred_element_type=jnp.float32)
    d2_ref[...] = sqt + sqf - 2.0 * cross

    rows = lax.broadcasted_iota(jnp.int32, (TR, KP), 0) + t * TR

    iotaf = lax.broadcasted_iota(jnp.int32, (TR, N), 1).astype(jnp.float32)
    kcol = lax.broadcasted_iota(jnp.int32, (TR, KP), 1)
    inf = jnp.float32(jnp.inf)
    nf = jnp.float32(N)

    def step(k, acc):
        v = d2_ref[...]
        m = jnp.min(v, axis=1, keepdims=True)
        sel = jnp.where(v == m, iotaf, nf)
        i = jnp.min(sel, axis=1, keepdims=True)
        acc = jnp.where(kcol == k, i.astype(jnp.int32), acc)
        d2_ref[...] = jnp.where(iotaf == i, inf, v)
        return acc

    idx_final = lax.fori_loop(0, K, step, rows)
    idx_ref[0] = idx_final + b * N

    wg = wg_ref[...]
    a = wg[:, 0:3]
    d = wg[:, 3:6]
    rgeo = lax.dot_general(xt, d, (((1,), (1,)), ((), ())),
                           preferred_element_type=jnp.float32)
    cgeo = lax.dot_general(xt, a - d, (((1,), (1,)), ((), ())),
                           preferred_element_type=jnp.float32)
    fb = f_ref[0]
    wf = wf_ref[...]
    w1 = wf[:, 0:C]
    w2 = wf[:, C:2 * C]
    rfeat = lax.dot_general(fb, w1, (((0,), (1,)), ((), ())),
                            preferred_element_type=jnp.float32)
    cfeat = lax.dot_general(fb, w2, (((0,), (1,)), ((), ())),
                            preferred_element_type=jnp.float32)
    r_ref[...] = jnp.concatenate([rgeo, rfeat], axis=1)
    c_ref[...] = jnp.concatenate([cgeo, cfeat], axis=1)


def _knn_proj(xyz, xyzt, f, w_geo, w_feat):
    nt = N // TR
    return pl.pallas_call(
        _knn_proj_body,
        grid=(B, nt),
        in_specs=[
            pl.BlockSpec((1, TR, 3), lambda b, t: (b, t, 0)),
            pl.BlockSpec((1, 3, N), lambda b, t: (b, 0, 0)),
            pl.BlockSpec((1, C, TR), lambda b, t: (b, 0, t)),
            pl.BlockSpec((OUT, 6), lambda b, t: (0, 0)),
            pl.BlockSpec((OUT, 2 * C), lambda b, t: (0, 0)),
        ],
        out_specs=[
            pl.BlockSpec((1, TR, KP), lambda b, t: (b, t, 0)),
            pl.BlockSpec((TR, NCH), lambda b, t: (b * nt + t, 0)),
            pl.BlockSpec((TR, NCH), lambda b, t: (b * nt + t, 0)),
        ],
        out_shape=[
            jax.ShapeDtypeStruct((B, N, KP), jnp.int32),
            jax.ShapeDtypeStruct((B * N, NCH), jnp.float32),
            jax.ShapeDtypeStruct((B * N, NCH), jnp.float32),
        ],
        scratch_shapes=[
            pltpu.VMEM((TR, N), jnp.float32),
        ],
    )(xyz, xyzt, f, w_geo, w_feat)


def _gather_reduce_body(table_ref, idxg_ref, ctab_ref, m_ref, stats_ref,
                        idx_v, rows_v, cbuf, mbuf, acc, sems):
    cid = lax.axis_index("c")
    sid = lax.axis_index("s")
    wid = sid * 2 + cid
    gbase = wid * NG
    pbase = wid * PW

    pltpu.sync_copy(idxg_ref.at[pl.ds(gbase, NG)], idx_v)

    zeros = jnp.zeros((16,), jnp.float32)
    for i in range(5):
        for cch in range(NCH // 16):
            acc[i, pl.ds(cch * 16, 16)] = zeros

    def process(g, buf):
        pltpu.sync_copy(ctab_ref.at[pl.ds(pbase + g * GP, GP)], cbuf)
        pltpu.make_async_copy(
            table_ref.at[idx_v.at[g]], rows_v.at[buf], sems.at[buf]).wait()

        for p in range(GP):
            base = p * KP
            m0 = [rows_v[buf, base, pl.ds(cch * 16, 16)]
                  for cch in range(NCH // 16)]
            s0 = list(m0)
            q0 = [v * v for v in m0]

            def kstep(k, carry):
                ms, ss, qs = carry
                nm, ns, nq = [], [], []
                for cch in range(NCH // 16):
                    v = rows_v[buf, base + k, pl.ds(cch * 16, 16)]
                    nm.append(jnp.maximum(ms[cch], v))
                    ns.append(ss[cch] + v)
                    nq.append(qs[cch] + v * v)
                return tuple(nm), tuple(ns), tuple(nq)

            ms, ss, qs = lax.fori_loop(
                1, K, kstep, (tuple(m0), tuple(s0), tuple(q0)))

            for cch in range(NCH // 16):
                sl = pl.ds(cch * 16, 16)
                cv = cbuf[p, sl]
                mbuf[p, sl] = cv + ms[cch]
                acc[0, sl] = acc[0, sl] + ss[cch]
                acc[1, sl] = acc[1, sl] + qs[cch]
                acc[2, sl] = acc[2, sl] + cv * ss[cch]
                acc[3, sl] = acc[3, sl] + cv
                acc[4, sl] = acc[4, sl] + cv * cv

        pltpu.sync_copy(mbuf, m_ref.at[pl.ds(pbase + g * GP, GP)])

    def fire(g, buf):
        pltpu.async_copy(table_ref.at[idx_v.at[g]], rows_v.at[buf], sems.at[buf])

    fire(0, 0)

    def pair(h, _):
        g0 = 2 * h
        fire(g0 + 1, 1)
        process(g0, 0)

        @pl.when(g0 + 2 < NG)
        def _():
            fire(g0 + 2, 0)

        process(g0 + 1, 1)
        return 0

    lax.fori_loop(0, NG // 2, pair, 0)
    pltpu.sync_copy(acc, stats_ref.at[wid])


def _gather_reduce(table, idxg, ctab):
    mesh = plsc.VectorSubcoreMesh(
        core_axis_name="c", subcore_axis_name="s",
        num_cores=2, num_subcores=16)
    kern = pl.kernel(
        _gather_reduce_body,
        out_type=[
            jax.ShapeDtypeStruct((B * N, NCH), jnp.float32),
            jax.ShapeDtypeStruct((NW, 5, NCH), jnp.float32),
        ],
        mesh=mesh,
        scratch_types=[
            pltpu.VMEM((NG, GI), jnp.int32),
            pltpu.VMEM((2, GI, NCH), jnp.float32),
            pltpu.VMEM((GP, NCH), jnp.float32),
            pltpu.VMEM((GP, NCH), jnp.float32),
            pltpu.VMEM((5, NCH), jnp.float32),
            pltpu.SemaphoreType.DMA((2,)),
        ],
    )
    return kern(table, idxg, ctab)


def _finalize_body(m_ref, stats_ref, g_ref, bt_ref, out_ref):
    st = stats_ref[...]
    tot = jnp.sum(st, axis=0)
    a1, a2, a3, a4, a5 = tot[0], tot[1], tot[2], tot[3], tot[4]
    kf = jnp.float32(K)
    inv = jnp.float32(1.0 / CNT)
    mean = (kf * a4 + a1) * inv
    e2 = (kf * a5 + 2.0 * a3 + a2) * inv
    var = e2 - mean * mean
    scale = g_ref[0] * lax.rsqrt(var + jnp.float32(EPS))
    shift = bt_ref[0] - mean * scale
    y = jnp.maximum(m_ref[...] * scale[None, :] + shift[None, :], 0.0)
    out_ref[0] = y.T


def _finalize(m, stats, gamma, beta):
    nt = N // TN
    return pl.pallas_call(
        _finalize_body,
        grid=(B, nt),
        in_specs=[
            pl.BlockSpec((TN, NCH), lambda b, t: (b * nt + t, 0)),
            pl.BlockSpec((NW, 5, NCH), lambda b, t: (0, 0, 0)),
            pl.BlockSpec((1, NCH), lambda b, t: (0, 0)),
            pl.BlockSpec((1, NCH), lambda b, t: (0, 0)),
        ],
        out_specs=pl.BlockSpec((1, NCH, TN), lambda b, t: (b, 0, t)),
        out_shape=jax.ShapeDtypeStruct((B, NCH, N), jnp.float32),
    )(m, stats, gamma, beta)


@jax.jit
def kernel(xyz, f, W_geo, gamma_geo, beta_geo, W_feat, gamma_feat, beta_feat):
    xyzt = jnp.transpose(xyz, (0, 2, 1))
    idx, rtab, ctab = _knn_proj(xyz, xyzt, f, W_geo, W_feat)
    idxg = idx.reshape((B * N) // GP, GI)
    m, stats = _gather_reduce(rtab, idxg, ctab)
    gamma = jnp.concatenate([gamma_geo, gamma_feat]).reshape(1, NCH)
    beta = jnp.concatenate([beta_geo, beta_feat]).reshape(1, NCH)
    return _finalize(m, stats, gamma, beta)


# speedup vs baseline: 1.3114x; 1.0475x over previous
import functools

import jax
import jax.numpy as jnp
from jax import lax
from jax.experimental import pallas as pl
from jax.experimental.pallas import tpu as pltpu
from jax.experimental.pallas import tpu_sc as plsc

B, N, C, OUT = 4, 4096, 64, 64
K = 21
KP = 24
TR = 256
TN = 512
NW = 32
BH = 2
PW = (BH * N) // NW
GP = 4
GI = GP * KP
NG = PW // GP
NCH = 2 * OUT
CNT = B * N * K
EPS = 1e-5


def _knn_proj_body(xyz_ref, xyzt_ref, f_ref, wg_ref, wf_ref,
                   idx_ref, r_ref, c_ref, d2_ref):
    b = pl.program_id(0)
    t = pl.program_id(1)

    xt = xyz_ref[0]
    xft = xyzt_ref[0]

    sqt = jnp.sum(xt * xt, axis=1, keepdims=True)
    sqf = jnp.sum(xft * xft, axis=0)[None, :]
    cross = lax.dot_general(xt, xft, (((1,), (0,)), ((), ())),
                            preferred_element_type=jnp.float32)
    d2_ref[...] = sqt + sqf - 2.0 * cross

    rows = lax.broadcasted_iota(jnp.int32, (TR, KP), 0) + t * TR

    iotaf = lax.broadcasted_iota(jnp.int32, (TR, N), 1).astype(jnp.float32)
    kcol = lax.broadcasted_iota(jnp.int32, (TR, KP), 1)
    inf = jnp.float32(jnp.inf)
    nf = jnp.float32(N)

    def step(k, acc):
        v = d2_ref[...]
        m = jnp.min(v, axis=1, keepdims=True)
        sel = jnp.where(v == m, iotaf, nf)
        i = jnp.min(sel, axis=1, keepdims=True)
        acc = jnp.where(kcol == k, i.astype(jnp.int32), acc)
        d2_ref[...] = jnp.where(iotaf == i, inf, v)
        return acc

    idx_final = lax.fori_loop(0, K, step, rows)
    idx_ref[0] = idx_final + b * N

    wg = wg_ref[...]
    a = wg[:, 0:3]
    d = wg[:, 3:6]
    rgeo = lax.dot_general(xt, d, (((1,), (1,)), ((), ())),
                           preferred_element_type=jnp.float32)
    cgeo = lax.dot_general(xt, a - d, (((1,), (1,)), ((), ())),
                           preferred_element_type=jnp.float32)
    fb = f_ref[0]
    wf = wf_ref[...]
    w1 = wf[:, 0:C]
    w2 = wf[:, C:2 * C]
    rfeat = lax.dot_general(fb, w1, (((0,), (1,)), ((), ())),
                            preferred_element_type=jnp.float32)
    cfeat = lax.dot_general(fb, w2, (((0,), (1,)), ((), ())),
                            preferred_element_type=jnp.float32)
    r_ref[...] = jnp.concatenate([rgeo, rfeat], axis=1)
    c_ref[...] = jnp.concatenate([cgeo, cfeat], axis=1)


def _knn_proj(xyz, xyzt, f, w_geo, w_feat, h):
    nt = N // TR
    return pl.pallas_call(
        _knn_proj_body,
        grid=(BH, nt),
        in_specs=[
            pl.BlockSpec((1, TR, 3), lambda b, t: (h * BH + b, t, 0)),
            pl.BlockSpec((1, 3, N), lambda b, t: (h * BH + b, 0, 0)),
            pl.BlockSpec((1, C, TR), lambda b, t: (h * BH + b, 0, t)),
            pl.BlockSpec((OUT, 6), lambda b, t: (0, 0)),
            pl.BlockSpec((OUT, 2 * C), lambda b, t: (0, 0)),
        ],
        out_specs=[
            pl.BlockSpec((1, TR, KP), lambda b, t: (b, t, 0)),
            pl.BlockSpec((TR, NCH), lambda b, t: (b * nt + t, 0)),
            pl.BlockSpec((TR, NCH), lambda b, t: (b * nt + t, 0)),
        ],
        out_shape=[
            jax.ShapeDtypeStruct((BH, N, KP), jnp.int32),
            jax.ShapeDtypeStruct((BH * N, NCH), jnp.float32),
            jax.ShapeDtypeStruct((BH * N, NCH), jnp.float32),
        ],
        scratch_shapes=[
            pltpu.VMEM((TR, N), jnp.float32),
        ],
    )(xyz, xyzt, f, w_geo, w_feat)


def _gather_reduce_body(table_ref, idxg_ref, ctab_ref, m_ref, stats_ref,
                        idx_v, rows_v, cbuf, mbuf, acc, sems):
    cid = lax.axis_index("c")
    sid = lax.axis_index("s")
    wid = sid * 2 + cid
    gbase = wid * NG
    pbase = wid * PW

    pltpu.sync_copy(idxg_ref.at[pl.ds(gbase, NG)], idx_v)

    zeros = jnp.zeros((16,), jnp.float32)
    for i in range(5):
        for cch in range(NCH // 16):
            acc[i, pl.ds(cch * 16, 16)] = zeros

    def process(g, buf):
        pltpu.sync_copy(ctab_ref.at[pl.ds(pbase + g * GP, GP)], cbuf)
        pltpu.make_async_copy(
            table_ref.at[idx_v.at[g]], rows_v.at[buf], sems.at[buf]).wait()

        for p in range(GP):
            base = p * KP
            m0 = [rows_v[buf, base, pl.ds(cch * 16, 16)]
                  for cch in range(NCH // 16)]
            s0 = list(m0)
            q0 = [v * v for v in m0]

            def kstep(k, carry):
                ms, ss, qs = carry
                nm, ns, nq = [], [], []
                for cch in range(NCH // 16):
                    v = rows_v[buf, base + k, pl.ds(cch * 16, 16)]
                    nm.append(jnp.maximum(ms[cch], v))
                    ns.append(ss[cch] + v)
                    nq.append(qs[cch] + v * v)
                return tuple(nm), tuple(ns), tuple(nq)

            ms, ss, qs = lax.fori_loop(
                1, K, kstep, (tuple(m0), tuple(s0), tuple(q0)))

            for cch in range(NCH // 16):
                sl = pl.ds(cch * 16, 16)
                cv = cbuf[p, sl]
                mbuf[p, sl] = cv + ms[cch]
                acc[0, sl] = acc[0, sl] + ss[cch]
                acc[1, sl] = acc[1, sl] + qs[cch]
                acc[2, sl] = acc[2, sl] + cv * ss[cch]
                acc[3, sl] = acc[3, sl] + cv
                acc[4, sl] = acc[4, sl] + cv * cv

        pltpu.sync_copy(mbuf, m_ref.at[pl.ds(pbase + g * GP, GP)])

    def fire(g, buf):
        pltpu.async_copy(table_ref.at[idx_v.at[g]], rows_v.at[buf], sems.at[buf])

    fire(0, 0)

    def pair(h, _):
        g0 = 2 * h
        fire(g0 + 1, 1)
        process(g0, 0)

        @pl.when(g0 + 2 < NG)
        def _():
            fire(g0 + 2, 0)

        process(g0 + 1, 1)
        return 0

    lax.fori_loop(0, NG // 2, pair, 0)
    pltpu.sync_copy(acc, stats_ref.at[wid])


def _gather_reduce(table, idxg, ctab):
    mesh = plsc.VectorSubcoreMesh(
        core_axis_name="c", subcore_axis_name="s",
        num_cores=2, num_subcores=16)
    kern = pl.kernel(
        _gather_reduce_body,
        out_type=[
            jax.ShapeDtypeStruct((BH * N, NCH), jnp.float32),
            jax.ShapeDtypeStruct((NW, 5, NCH), jnp.float32),
        ],
        mesh=mesh,
        scratch_types=[
            pltpu.VMEM((NG, GI), jnp.int32),
            pltpu.VMEM((2, GI, NCH), jnp.float32),
            pltpu.VMEM((GP, NCH), jnp.float32),
            pltpu.VMEM((GP, NCH), jnp.float32),
            pltpu.VMEM((5, NCH), jnp.float32),
            pltpu.SemaphoreType.DMA((2,)),
        ],
    )
    return kern(table, idxg, ctab)


def _finalize_body(m_ref, stats_ref, g_ref, bt_ref, out_ref):
    st = stats_ref[...]
    tot = jnp.sum(st, axis=0)
    a1, a2, a3, a4, a5 = tot[0], tot[1], tot[2], tot[3], tot[4]
    kf = jnp.float32(K)
    inv = jnp.float32(1.0 / CNT)
    mean = (kf * a4 + a1) * inv
    e2 = (kf * a5 + 2.0 * a3 + a2) * inv
    var = e2 - mean * mean
    scale = g_ref[0] * lax.rsqrt(var + jnp.float32(EPS))
    shift = bt_ref[0] - mean * scale
    y = jnp.maximum(m_ref[...] * scale[None, :] + shift[None, :], 0.0)
    out_ref[0] = y.T


def _finalize(m, stats, gamma, beta):
    nt = N // TN
    return pl.pallas_call(
        _finalize_body,
        grid=(B, nt),
        in_specs=[
            pl.BlockSpec((TN, NCH), lambda b, t: (b * nt + t, 0)),
            pl.BlockSpec((2 * NW, 5, NCH), lambda b, t: (0, 0, 0)),
            pl.BlockSpec((1, NCH), lambda b, t: (0, 0)),
            pl.BlockSpec((1, NCH), lambda b, t: (0, 0)),
        ],
        out_specs=pl.BlockSpec((1, NCH, TN), lambda b, t: (b, 0, t)),
        out_shape=jax.ShapeDtypeStruct((B, NCH, N), jnp.float32),
    )(m, stats, gamma, beta)


@jax.jit
def kernel(xyz, f, W_geo, gamma_geo, beta_geo, W_feat, gamma_feat, beta_feat):
    xyzt = jnp.transpose(xyz, (0, 2, 1))
    halves = []
    for h in range(B // BH):
        idx, rtab, ctab = _knn_proj(xyz, xyzt, f, W_geo, W_feat, h)
        idxg = idx.reshape((BH * N) // GP, GI)
        halves.append((rtab, idxg, ctab))
    ms, sts = [], []
    for rtab, idxg, ctab in halves:
        m, stats = _gather_reduce(rtab, idxg, ctab)
        ms.append(m)
        sts.append(stats)
    m = jnp.concatenate(ms, axis=0)
    stats = jnp.concatenate(sts, axis=0)
    gamma = jnp.concatenate([gamma_geo, gamma_feat]).reshape(1, NCH)
    beta = jnp.concatenate([beta_geo, beta_feat]).reshape(1, NCH)
    return _finalize(m, stats, gamma, beta)


# speedup vs baseline: 1.3433x; 1.0243x over previous
import functools

import jax
import jax.numpy as jnp
from jax import lax
from jax.experimental import pallas as pl
from jax.experimental.pallas import tpu as pltpu
from jax.experimental.pallas import tpu_sc as plsc

B, N, C, OUT = 4, 4096, 64, 64
K = 21
KP = 24
TR = 256
TN = 512
NW = 32
BH = 1
PW = (BH * N) // NW
GP = 4
GI = GP * KP
NG = PW // GP
NCH = 2 * OUT
CNT = B * N * K
EPS = 1e-5


def _knn_proj_body(xyz_ref, xyzt_ref, f_ref, wg_ref, wf_ref,
                   idx_ref, r_ref, c_ref, d2_ref):
    b = pl.program_id(0)
    t = pl.program_id(1)

    xt = xyz_ref[0]
    xft = xyzt_ref[0]

    sqt = jnp.sum(xt * xt, axis=1, keepdims=True)
    sqf = jnp.sum(xft * xft, axis=0)[None, :]
    cross = lax.dot_general(xt, xft, (((1,), (0,)), ((), ())),
                            preferred_element_type=jnp.float32)
    d2_ref[...] = sqt + sqf - 2.0 * cross

    rows = lax.broadcasted_iota(jnp.int32, (TR, KP), 0) + t * TR

    iotaf = lax.broadcasted_iota(jnp.int32, (TR, N), 1).astype(jnp.float32)
    kcol = lax.broadcasted_iota(jnp.int32, (TR, KP), 1)
    inf = jnp.float32(jnp.inf)
    nf = jnp.float32(N)

    def step(k, acc):
        v = d2_ref[...]
        m = jnp.min(v, axis=1, keepdims=True)
        sel = jnp.where(v == m, iotaf, nf)
        i = jnp.min(sel, axis=1, keepdims=True)
        acc = jnp.where(kcol == k, i.astype(jnp.int32), acc)
        d2_ref[...] = jnp.where(iotaf == i, inf, v)
        return acc

    idx_final = lax.fori_loop(0, K, step, rows)
    idx_ref[0] = idx_final + b * N

    wg = wg_ref[...]
    a = wg[:, 0:3]
    d = wg[:, 3:6]
    rgeo = lax.dot_general(xt, d, (((1,), (1,)), ((), ())),
                           preferred_element_type=jnp.float32)
    cgeo = lax.dot_general(xt, a - d, (((1,), (1,)), ((), ())),
                           preferred_element_type=jnp.float32)
    fb = f_ref[0]
    wf = wf_ref[...]
    w1 = wf[:, 0:C]
    w2 = wf[:, C:2 * C]
    rfeat = lax.dot_general(fb, w1, (((0,), (1,)), ((), ())),
                            preferred_element_type=jnp.float32)
    cfeat = lax.dot_general(fb, w2, (((0,), (1,)), ((), ())),
                            preferred_element_type=jnp.float32)
    r_ref[...] = jnp.concatenate([rgeo, rfeat], axis=1)
    c_ref[...] = jnp.concatenate([cgeo, cfeat], axis=1)


def _knn_proj(xyz, xyzt, f, w_geo, w_feat, h):
    nt = N // TR
    return pl.pallas_call(
        _knn_proj_body,
        grid=(BH, nt),
        in_specs=[
            pl.BlockSpec((1, TR, 3), lambda b, t: (h * BH + b, t, 0)),
            pl.BlockSpec((1, 3, N), lambda b, t: (h * BH + b, 0, 0)),
            pl.BlockSpec((1, C, TR), lambda b, t: (h * BH + b, 0, t)),
            pl.BlockSpec((OUT, 6), lambda b, t: (0, 0)),
            pl.BlockSpec((OUT, 2 * C), lambda b, t: (0, 0)),
        ],
        out_specs=[
            pl.BlockSpec((1, TR, KP), lambda b, t: (b, t, 0)),
            pl.BlockSpec((TR, NCH), lambda b, t: (b * nt + t, 0)),
            pl.BlockSpec((TR, NCH), lambda b, t: (b * nt + t, 0)),
        ],
        out_shape=[
            jax.ShapeDtypeStruct((BH, N, KP), jnp.int32),
            jax.ShapeDtypeStruct((BH * N, NCH), jnp.float32),
            jax.ShapeDtypeStruct((BH * N, NCH), jnp.float32),
        ],
        scratch_shapes=[
            pltpu.VMEM((TR, N), jnp.float32),
        ],
    )(xyz, xyzt, f, w_geo, w_feat)


def _gather_reduce_body(table_ref, idxg_ref, ctab_ref, m_ref, stats_ref,
                        idx_v, rows_v, cbuf, mbuf, acc, sems):
    cid = lax.axis_index("c")
    sid = lax.axis_index("s")
    wid = sid * 2 + cid
    gbase = wid * NG
    pbase = wid * PW

    pltpu.sync_copy(idxg_ref.at[pl.ds(gbase, NG)], idx_v)

    zeros = jnp.zeros((16,), jnp.float32)
    for i in range(5):
        for cch in range(NCH // 16):
            acc[i, pl.ds(cch * 16, 16)] = zeros

    def process(g, buf):
        pltpu.sync_copy(ctab_ref.at[pl.ds(pbase + g * GP, GP)], cbuf)
        pltpu.make_async_copy(
            table_ref.at[idx_v.at[g]], rows_v.at[buf], sems.at[buf]).wait()

        for p in range(GP):
            base = p * KP
            m0 = [rows_v[buf, base, pl.ds(cch * 16, 16)]
                  for cch in range(NCH // 16)]
            s0 = list(m0)
            q0 = [v * v for v in m0]

            def kstep(k, carry):
                ms, ss, qs = carry
                nm, ns, nq = [], [], []
                for cch in range(NCH // 16):
                    v = rows_v[buf, base + k, pl.ds(cch * 16, 16)]
                    nm.append(jnp.maximum(ms[cch], v))
                    ns.append(ss[cch] + v)
                    nq.append(qs[cch] + v * v)
                return tuple(nm), tuple(ns), tuple(nq)

            ms, ss, qs = lax.fori_loop(
                1, K, kstep, (tuple(m0), tuple(s0), tuple(q0)))

            for cch in range(NCH // 16):
                sl = pl.ds(cch * 16, 16)
                cv = cbuf[p, sl]
                mbuf[p, sl] = cv + ms[cch]
                acc[0, sl] = acc[0, sl] + ss[cch]
                acc[1, sl] = acc[1, sl] + qs[cch]
                acc[2, sl] = acc[2, sl] + cv * ss[cch]
                acc[3, sl] = acc[3, sl] + cv
                acc[4, sl] = acc[4, sl] + cv * cv

        pltpu.sync_copy(mbuf, m_ref.at[pl.ds(pbase + g * GP, GP)])

    def fire(g, buf):
        pltpu.async_copy(table_ref.at[idx_v.at[g]], rows_v.at[buf], sems.at[buf])

    fire(0, 0)

    def pair(h, _):
        g0 = 2 * h
        fire(g0 + 1, 1)
        process(g0, 0)

        @pl.when(g0 + 2 < NG)
        def _():
            fire(g0 + 2, 0)

        process(g0 + 1, 1)
        return 0

    lax.fori_loop(0, NG // 2, pair, 0)
    pltpu.sync_copy(acc, stats_ref.at[wid])


def _gather_reduce(table, idxg, ctab):
    mesh = plsc.VectorSubcoreMesh(
        core_axis_name="c", subcore_axis_name="s",
        num_cores=2, num_subcores=16)
    kern = pl.kernel(
        _gather_reduce_body,
        out_type=[
            jax.ShapeDtypeStruct((BH * N, NCH), jnp.float32),
            jax.ShapeDtypeStruct((NW, 5, NCH), jnp.float32),
        ],
        mesh=mesh,
        scratch_types=[
            pltpu.VMEM((NG, GI), jnp.int32),
            pltpu.VMEM((2, GI, NCH), jnp.float32),
            pltpu.VMEM((GP, NCH), jnp.float32),
            pltpu.VMEM((GP, NCH), jnp.float32),
            pltpu.VMEM((5, NCH), jnp.float32),
            pltpu.SemaphoreType.DMA((2,)),
        ],
    )
    return kern(table, idxg, ctab)


def _finalize_body(m_ref, stats_ref, g_ref, bt_ref, out_ref):
    st = stats_ref[...]
    tot = jnp.sum(st, axis=0)
    a1, a2, a3, a4, a5 = tot[0], tot[1], tot[2], tot[3], tot[4]
    kf = jnp.float32(K)
    inv = jnp.float32(1.0 / CNT)
    mean = (kf * a4 + a1) * inv
    e2 = (kf * a5 + 2.0 * a3 + a2) * inv
    var = e2 - mean * mean
    scale = g_ref[0] * lax.rsqrt(var + jnp.float32(EPS))
    shift = bt_ref[0] - mean * scale
    y = jnp.maximum(m_ref[...] * scale[None, :] + shift[None, :], 0.0)
    out_ref[0] = y.T


def _finalize(m, stats, gamma, beta):
    nt = N // TN
    return pl.pallas_call(
        _finalize_body,
        grid=(B, nt),
        in_specs=[
            pl.BlockSpec((TN, NCH), lambda b, t: (b * nt + t, 0)),
            pl.BlockSpec(((B // BH) * NW, 5, NCH), lambda b, t: (0, 0, 0)),
            pl.BlockSpec((1, NCH), lambda b, t: (0, 0)),
            pl.BlockSpec((1, NCH), lambda b, t: (0, 0)),
        ],
        out_specs=pl.BlockSpec((1, NCH, TN), lambda b, t: (b, 0, t)),
        out_shape=jax.ShapeDtypeStruct((B, NCH, N), jnp.float32),
    )(m, stats, gamma, beta)


@jax.jit
def kernel(xyz, f, W_geo, gamma_geo, beta_geo, W_feat, gamma_feat, beta_feat):
    xyzt = jnp.transpose(xyz, (0, 2, 1))
    halves = []
    for h in range(B // BH):
        idx, rtab, ctab = _knn_proj(xyz, xyzt, f, W_geo, W_feat, h)
        idxg = idx.reshape((BH * N) // GP, GI)
        halves.append((rtab, idxg, ctab))
    ms, sts = [], []
    for rtab, idxg, ctab in halves:
        m, stats = _gather_reduce(rtab, idxg, ctab)
        ms.append(m)
        sts.append(stats)
    m = jnp.concatenate(ms, axis=0)
    stats = jnp.concatenate(sts, axis=0)
    gamma = jnp.concatenate([gamma_geo, gamma_feat]).reshape(1, NCH)
    beta = jnp.concatenate([beta_geo, beta_feat]).reshape(1, NCH)
    return _finalize(m, stats, gamma, beta)


# speedup vs baseline: 1.4132x; 1.0520x over previous
import functools

import jax
import jax.numpy as jnp
from jax import lax
from jax.experimental import pallas as pl
from jax.experimental.pallas import tpu as pltpu
from jax.experimental.pallas import tpu_sc as plsc

B, N, C, OUT = 4, 4096, 64, 64
K = 21
KP = 24
TR = 512
TN = 512
NW = 32
BH = 1
PW = (BH * N) // NW
GP = 4
GI = GP * KP
NG = PW // GP
NCH = 2 * OUT
CNT = B * N * K
EPS = 1e-5


def _knn_proj_body(xyz_ref, xyzt_ref, f_ref, wg_ref, wf_ref,
                   idx_ref, r_ref, c_ref, d2_ref):
    b = pl.program_id(0)
    t = pl.program_id(1)

    xt = xyz_ref[0]
    xft = xyzt_ref[0]

    sqt = jnp.sum(xt * xt, axis=1, keepdims=True)
    sqf = jnp.sum(xft * xft, axis=0)[None, :]
    cross = lax.dot_general(xt, xft, (((1,), (0,)), ((), ())),
                            preferred_element_type=jnp.float32)
    d2_ref[...] = sqt + sqf - 2.0 * cross

    rows = lax.broadcasted_iota(jnp.int32, (TR, KP), 0) + t * TR

    iotaf = lax.broadcasted_iota(jnp.int32, (TR, N), 1).astype(jnp.float32)
    kcol = lax.broadcasted_iota(jnp.int32, (TR, KP), 1)
    inf = jnp.float32(jnp.inf)
    nf = jnp.float32(N)

    def step(k, acc):
        v = d2_ref[...]
        m = jnp.min(v, axis=1, keepdims=True)
        sel = jnp.where(v == m, iotaf, nf)
        i = jnp.min(sel, axis=1, keepdims=True)
        acc = jnp.where(kcol == k, i.astype(jnp.int32), acc)
        d2_ref[...] = jnp.where(iotaf == i, inf, v)
        return acc

    idx_final = lax.fori_loop(0, K, step, rows, unroll=3)
    idx_ref[0] = idx_final + b * N

    wg = wg_ref[...]
    a = wg[:, 0:3]
    d = wg[:, 3:6]
    rgeo = lax.dot_general(xt, d, (((1,), (1,)), ((), ())),
                           preferred_element_type=jnp.float32)
    cgeo = lax.dot_general(xt, a - d, (((1,), (1,)), ((), ())),
                           preferred_element_type=jnp.float32)
    fb = f_ref[0]
    wf = wf_ref[...]
    w1 = wf[:, 0:C]
    w2 = wf[:, C:2 * C]
    rfeat = lax.dot_general(fb, w1, (((0,), (1,)), ((), ())),
                            preferred_element_type=jnp.float32)
    cfeat = lax.dot_general(fb, w2, (((0,), (1,)), ((), ())),
                            preferred_element_type=jnp.float32)
    r_ref[...] = jnp.concatenate([rgeo, rfeat], axis=1)
    c_ref[...] = jnp.concatenate([cgeo, cfeat], axis=1)


def _knn_proj(xyz, xyzt, f, w_geo, w_feat, h):
    nt = N // TR
    return pl.pallas_call(
        _knn_proj_body,
        grid=(BH, nt),
        in_specs=[
            pl.BlockSpec((1, TR, 3), lambda b, t: (h * BH + b, t, 0)),
            pl.BlockSpec((1, 3, N), lambda b, t: (h * BH + b, 0, 0)),
            pl.BlockSpec((1, C, TR), lambda b, t: (h * BH + b, 0, t)),
            pl.BlockSpec((OUT, 6), lambda b, t: (0, 0)),
            pl.BlockSpec((OUT, 2 * C), lambda b, t: (0, 0)),
        ],
        out_specs=[
            pl.BlockSpec((1, TR, KP), lambda b, t: (b, t, 0)),
            pl.BlockSpec((TR, NCH), lambda b, t: (b * nt + t, 0)),
            pl.BlockSpec((TR, NCH), lambda b, t: (b * nt + t, 0)),
        ],
        out_shape=[
            jax.ShapeDtypeStruct((BH, N, KP), jnp.int32),
            jax.ShapeDtypeStruct((BH * N, NCH), jnp.float32),
            jax.ShapeDtypeStruct((BH * N, NCH), jnp.float32),
        ],
        scratch_shapes=[
            pltpu.VMEM((TR, N), jnp.float32),
        ],
    )(xyz, xyzt, f, w_geo, w_feat)


def _gather_reduce_body(table_ref, idxg_ref, ctab_ref, m_ref, stats_ref,
                        idx_v, rows_v, cbuf, mbuf, acc, sems):
    cid = lax.axis_index("c")
    sid = lax.axis_index("s")
    wid = sid * 2 + cid
    gbase = wid * NG
    pbase = wid * PW

    pltpu.sync_copy(idxg_ref.at[pl.ds(gbase, NG)], idx_v)

    zeros = jnp.zeros((16,), jnp.float32)
    for i in range(5):
        for cch in range(NCH // 16):
            acc[i, pl.ds(cch * 16, 16)] = zeros

    def process(g, buf):
        pltpu.sync_copy(ctab_ref.at[pl.ds(pbase + g * GP, GP)], cbuf)
        pltpu.make_async_copy(
            table_ref.at[idx_v.at[g]], rows_v.at[buf], sems.at[buf]).wait()

        for p in range(GP):
            base = p * KP
            m0 = [rows_v[buf, base, pl.ds(cch * 16, 16)]
                  for cch in range(NCH // 16)]
            s0 = list(m0)
            q0 = [v * v for v in m0]

            def kstep(k, carry):
                ms, ss, qs = carry
                nm, ns, nq = [], [], []
                for cch in range(NCH // 16):
                    v = rows_v[buf, base + k, pl.ds(cch * 16, 16)]
                    nm.append(jnp.maximum(ms[cch], v))
                    ns.append(ss[cch] + v)
                    nq.append(qs[cch] + v * v)
                return tuple(nm), tuple(ns), tuple(nq)

            ms, ss, qs = lax.fori_loop(
                1, K, kstep, (tuple(m0), tuple(s0), tuple(q0)))

            for cch in range(NCH // 16):
                sl = pl.ds(cch * 16, 16)
                cv = cbuf[p, sl]
                mbuf[p, sl] = cv + ms[cch]
                acc[0, sl] = acc[0, sl] + ss[cch]
                acc[1, sl] = acc[1, sl] + qs[cch]
                acc[2, sl] = acc[2, sl] + cv * ss[cch]
                acc[3, sl] = acc[3, sl] + cv
                acc[4, sl] = acc[4, sl] + cv * cv

        pltpu.sync_copy(mbuf, m_ref.at[pl.ds(pbase + g * GP, GP)])

    def fire(g, buf):
        pltpu.async_copy(table_ref.at[idx_v.at[g]], rows_v.at[buf], sems.at[buf])

    fire(0, 0)

    def pair(h, _):
        g0 = 2 * h
        fire(g0 + 1, 1)
        process(g0, 0)

        @pl.when(g0 + 2 < NG)
        def _():
            fire(g0 + 2, 0)

        process(g0 + 1, 1)
        return 0

    lax.fori_loop(0, NG // 2, pair, 0)
    pltpu.sync_copy(acc, stats_ref.at[wid])


def _gather_reduce(table, idxg, ctab):
    mesh = plsc.VectorSubcoreMesh(
        core_axis_name="c", subcore_axis_name="s",
        num_cores=2, num_subcores=16)
    kern = pl.kernel(
        _gather_reduce_body,
        out_type=[
            jax.ShapeDtypeStruct((BH * N, NCH), jnp.float32),
            jax.ShapeDtypeStruct((NW, 5, NCH), jnp.float32),
        ],
        mesh=mesh,
        scratch_types=[
            pltpu.VMEM((NG, GI), jnp.int32),
            pltpu.VMEM((2, GI, NCH), jnp.float32),
            pltpu.VMEM((GP, NCH), jnp.float32),
            pltpu.VMEM((GP, NCH), jnp.float32),
            pltpu.VMEM((5, NCH), jnp.float32),
            pltpu.SemaphoreType.DMA((2,)),
        ],
    )
    return kern(table, idxg, ctab)


def _finalize_body(m_ref, stats_ref, g_ref, bt_ref, out_ref):
    st = stats_ref[...]
    tot = jnp.sum(st, axis=0)
    a1, a2, a3, a4, a5 = tot[0], tot[1], tot[2], tot[3], tot[4]
    kf = jnp.float32(K)
    inv = jnp.float32(1.0 / CNT)
    mean = (kf * a4 + a1) * inv
    e2 = (kf * a5 + 2.0 * a3 + a2) * inv
    var = e2 - mean * mean
    scale = g_ref[0] * lax.rsqrt(var + jnp.float32(EPS))
    shift = bt_ref[0] - mean * scale
    y = jnp.maximum(m_ref[...] * scale[None, :] + shift[None, :], 0.0)
    out_ref[0] = y.T


def _finalize(m, stats, gamma, beta):
    nt = N // TN
    return pl.pallas_call(
        _finalize_body,
        grid=(B, nt),
        in_specs=[
            pl.BlockSpec((TN, NCH), lambda b, t: (b * nt + t, 0)),
            pl.BlockSpec(((B // BH) * NW, 5, NCH), lambda b, t: (0, 0, 0)),
            pl.BlockSpec((1, NCH), lambda b, t: (0, 0)),
            pl.BlockSpec((1, NCH), lambda b, t: (0, 0)),
        ],
        out_specs=pl.BlockSpec((1, NCH, TN), lambda b, t: (b, 0, t)),
        out_shape=jax.ShapeDtypeStruct((B, NCH, N), jnp.float32),
    )(m, stats, gamma, beta)


@jax.jit
def kernel(xyz, f, W_geo, gamma_geo, beta_geo, W_feat, gamma_feat, beta_feat):
    xyzt = jnp.transpose(xyz, (0, 2, 1))
    halves = []
    for h in range(B // BH):
        idx, rtab, ctab = _knn_proj(xyz, xyzt, f, W_geo, W_feat, h)
        idxg = idx.reshape((BH * N) // GP, GI)
        halves.append((rtab, idxg, ctab))
    ms, sts = [], []
    for rtab, idxg, ctab in halves:
        m, stats = _gather_reduce(rtab, idxg, ctab)
        ms.append(m)
        sts.append(stats)
    m = jnp.concatenate(ms, axis=0)
    stats = jnp.concatenate(sts, axis=0)
    gamma = jnp.concatenate([gamma_geo, gamma_feat]).reshape(1, NCH)
    beta = jnp.concatenate([beta_geo, beta_feat]).reshape(1, NCH)
    return _finalize(m, stats, gamma, beta)


# speedup vs baseline: 1.4557x; 1.0301x over previous
import functools

import jax
import jax.numpy as jnp
from jax import lax
from jax.experimental import pallas as pl
from jax.experimental.pallas import tpu as pltpu
from jax.experimental.pallas import tpu_sc as plsc

B, N, C, OUT = 4, 4096, 64, 64
K = 21
KP = 24
TR = 512
TN = 512
NW = 32
BH = 1
PW = (BH * N) // NW
GP = 4
GI = GP * KP
NG = PW // GP
NCH = 2 * OUT
CNT = B * N * K
EPS = 1e-5


def _knn_proj_body(xyz_ref, xyzt_ref, f_ref, wg_ref, wf_ref,
                   idx_ref, r_ref, c_ref, d2_ref):
    b = pl.program_id(0)
    t = pl.program_id(1)

    xt = xyz_ref[0]
    xft = xyzt_ref[0]

    sqt = jnp.sum(xt * xt, axis=1, keepdims=True)
    sqf = jnp.sum(xft * xft, axis=0)[None, :]
    cross = lax.dot_general(xt, xft, (((1,), (0,)), ((), ())),
                            preferred_element_type=jnp.float32)
    d2_ref[...] = sqt + sqf - 2.0 * cross

    rows = lax.broadcasted_iota(jnp.int32, (TR, KP), 0) + t * TR

    iotaf = lax.broadcasted_iota(jnp.int32, (TR, N), 1).astype(jnp.float32)
    kcol = lax.broadcasted_iota(jnp.int32, (TR, KP), 1)
    inf = jnp.float32(jnp.inf)
    nf = jnp.float32(N)

    def step(k, acc):
        v = d2_ref[...]
        m = jnp.min(v, axis=1, keepdims=True)
        sel = jnp.where(v == m, iotaf, nf)
        i = jnp.min(sel, axis=1, keepdims=True)
        acc = jnp.where(kcol == k, i.astype(jnp.int32), acc)
        d2_ref[...] = jnp.where(iotaf == i, inf, v)
        return acc

    idx_final = lax.fori_loop(0, K, step, rows, unroll=7)
    idx_ref[0] = idx_final + b * N

    wg = wg_ref[...]
    a = wg[:, 0:3]
    d = wg[:, 3:6]
    rgeo = lax.dot_general(xt, d, (((1,), (1,)), ((), ())),
                           preferred_element_type=jnp.float32)
    cgeo = lax.dot_general(xt, a - d, (((1,), (1,)), ((), ())),
                           preferred_element_type=jnp.float32)
    fb = f_ref[0]
    wf = wf_ref[...]
    w1 = wf[:, 0:C]
    w2 = wf[:, C:2 * C]
    rfeat = lax.dot_general(fb, w1, (((0,), (1,)), ((), ())),
                            preferred_element_type=jnp.float32)
    cfeat = lax.dot_general(fb, w2, (((0,), (1,)), ((), ())),
                            preferred_element_type=jnp.float32)
    r_ref[...] = jnp.concatenate([rgeo, rfeat], axis=1)
    c_ref[...] = jnp.concatenate([cgeo, cfeat], axis=1)


def _knn_proj(xyz, xyzt, f, w_geo, w_feat, h):
    nt = N // TR
    return pl.pallas_call(
        _knn_proj_body,
        grid=(BH, nt),
        in_specs=[
            pl.BlockSpec((1, TR, 3), lambda b, t: (h * BH + b, t, 0)),
            pl.BlockSpec((1, 3, N), lambda b, t: (h * BH + b, 0, 0)),
            pl.BlockSpec((1, C, TR), lambda b, t: (h * BH + b, 0, t)),
            pl.BlockSpec((OUT, 6), lambda b, t: (0, 0)),
            pl.BlockSpec((OUT, 2 * C), lambda b, t: (0, 0)),
        ],
        out_specs=[
            pl.BlockSpec((1, TR, KP), lambda b, t: (b, t, 0)),
            pl.BlockSpec((TR, NCH), lambda b, t: (b * nt + t, 0)),
            pl.BlockSpec((TR, NCH), lambda b, t: (b * nt + t, 0)),
        ],
        out_shape=[
            jax.ShapeDtypeStruct((BH, N, KP), jnp.int32),
            jax.ShapeDtypeStruct((BH * N, NCH), jnp.float32),
            jax.ShapeDtypeStruct((BH * N, NCH), jnp.float32),
        ],
        scratch_shapes=[
            pltpu.VMEM((TR, N), jnp.float32),
        ],
    )(xyz, xyzt, f, w_geo, w_feat)


def _gather_reduce_body(table_ref, idxg_ref, ctab_ref, m_ref, stats_ref,
                        idx_v, rows_v, cbuf, mbuf, acc, sems):
    cid = lax.axis_index("c")
    sid = lax.axis_index("s")
    wid = sid * 2 + cid
    gbase = wid * NG
    pbase = wid * PW

    pltpu.sync_copy(idxg_ref.at[pl.ds(gbase, NG)], idx_v)

    zeros = jnp.zeros((16,), jnp.float32)
    for i in range(5):
        for cch in range(NCH // 16):
            acc[i, pl.ds(cch * 16, 16)] = zeros

    def process(g, buf):
        pltpu.sync_copy(ctab_ref.at[pl.ds(pbase + g * GP, GP)], cbuf)
        pltpu.make_async_copy(
            table_ref.at[idx_v.at[g]], rows_v.at[buf], sems.at[buf]).wait()

        for p in range(GP):
            base = p * KP
            m0 = [rows_v[buf, base, pl.ds(cch * 16, 16)]
                  for cch in range(NCH // 16)]
            s0 = list(m0)
            q0 = [v * v for v in m0]

            def kstep(k, carry):
                ms, ss, qs = carry
                nm, ns, nq = [], [], []
                for cch in range(NCH // 16):
                    v = rows_v[buf, base + k, pl.ds(cch * 16, 16)]
                    nm.append(jnp.maximum(ms[cch], v))
                    ns.append(ss[cch] + v)
                    nq.append(qs[cch] + v * v)
                return tuple(nm), tuple(ns), tuple(nq)

            ms, ss, qs = lax.fori_loop(
                1, K, kstep, (tuple(m0), tuple(s0), tuple(q0)))

            for cch in range(NCH // 16):
                sl = pl.ds(cch * 16, 16)
                cv = cbuf[p, sl]
                mbuf[p, sl] = cv + ms[cch]
                acc[0, sl] = acc[0, sl] + ss[cch]
                acc[1, sl] = acc[1, sl] + qs[cch]
                acc[2, sl] = acc[2, sl] + cv * ss[cch]
                acc[3, sl] = acc[3, sl] + cv
                acc[4, sl] = acc[4, sl] + cv * cv

        pltpu.sync_copy(mbuf, m_ref.at[pl.ds(pbase + g * GP, GP)])

    def fire(g, buf):
        pltpu.async_copy(table_ref.at[idx_v.at[g]], rows_v.at[buf], sems.at[buf])

    fire(0, 0)

    def pair(h, _):
        g0 = 2 * h
        fire(g0 + 1, 1)
        process(g0, 0)

        @pl.when(g0 + 2 < NG)
        def _():
            fire(g0 + 2, 0)

        process(g0 + 1, 1)
        return 0

    lax.fori_loop(0, NG // 2, pair, 0)
    pltpu.sync_copy(acc, stats_ref.at[wid])


def _gather_reduce(table, idxg, ctab):
    mesh = plsc.VectorSubcoreMesh(
        core_axis_name="c", subcore_axis_name="s",
        num_cores=2, num_subcores=16)
    kern = pl.kernel(
        _gather_reduce_body,
        out_type=[
            jax.ShapeDtypeStruct((BH * N, NCH), jnp.float32),
            jax.ShapeDtypeStruct((NW, 5, NCH), jnp.float32),
        ],
        mesh=mesh,
        scratch_types=[
            pltpu.VMEM((NG, GI), jnp.int32),
            pltpu.VMEM((2, GI, NCH), jnp.float32),
            pltpu.VMEM((GP, NCH), jnp.float32),
            pltpu.VMEM((GP, NCH), jnp.float32),
            pltpu.VMEM((5, NCH), jnp.float32),
            pltpu.SemaphoreType.DMA((2,)),
        ],
    )
    return kern(table, idxg, ctab)


def _finalize_body(m_ref, stats_ref, g_ref, bt_ref, out_ref):
    st = stats_ref[...]
    tot = jnp.sum(st, axis=0)
    a1, a2, a3, a4, a5 = tot[0], tot[1], tot[2], tot[3], tot[4]
    kf = jnp.float32(K)
    inv = jnp.float32(1.0 / CNT)
    mean = (kf * a4 + a1) * inv
    e2 = (kf * a5 + 2.0 * a3 + a2) * inv
    var = e2 - mean * mean
    scale = g_ref[0] * lax.rsqrt(var + jnp.float32(EPS))
    shift = bt_ref[0] - mean * scale
    y = jnp.maximum(m_ref[...] * scale[None, :] + shift[None, :], 0.0)
    out_ref[0] = y.T


def _finalize(m, stats, gamma, beta):
    nt = N // TN
    return pl.pallas_call(
        _finalize_body,
        grid=(B, nt),
        in_specs=[
            pl.BlockSpec((TN, NCH), lambda b, t: (b * nt + t, 0)),
            pl.BlockSpec(((B // BH) * NW, 5, NCH), lambda b, t: (0, 0, 0)),
            pl.BlockSpec((1, NCH), lambda b, t: (0, 0)),
            pl.BlockSpec((1, NCH), lambda b, t: (0, 0)),
        ],
        out_specs=pl.BlockSpec((1, NCH, TN), lambda b, t: (b, 0, t)),
        out_shape=jax.ShapeDtypeStruct((B, NCH, N), jnp.float32),
    )(m, stats, gamma, beta)


@jax.jit
def kernel(xyz, f, W_geo, gamma_geo, beta_geo, W_feat, gamma_feat, beta_feat):
    xyzt = jnp.transpose(xyz, (0, 2, 1))
    halves = []
    for h in range(B // BH):
        idx, rtab, ctab = _knn_proj(xyz, xyzt, f, W_geo, W_feat, h)
        idxg = idx.reshape((BH * N) // GP, GI)
        halves.append((rtab, idxg, ctab))
    ms, sts = [], []
    for rtab, idxg, ctab in halves:
        m, stats = _gather_reduce(rtab, idxg, ctab)
        ms.append(m)
        sts.append(stats)
    m = jnp.concatenate(ms, axis=0)
    stats = jnp.concatenate(sts, axis=0)
    gamma = jnp.concatenate([gamma_geo, gamma_feat]).reshape(1, NCH)
    beta = jnp.concatenate([beta_geo, beta_feat]).reshape(1, NCH)
    return _finalize(m, stats, gamma, beta)
